# Initial kernel scaffold; baseline (speedup 1.0000x reference)
import jax
import jax.numpy as jnp
from jax.experimental import pallas as pl


def kernel(indices, tables, W1, b1, W2, b2):
    raise NotImplementedError("write your pallas kernel here")


# speedup vs baseline: 2.1762x; 2.1762x over previous
import functools

import jax
import jax.numpy as jnp
from jax import lax
from jax.experimental import pallas as pl
from jax.experimental.pallas import tpu as pltpu
from jax.experimental.pallas import tpu_sc as plsc


def _make_sc_gather(N, D, V, F, NC, NS):
    NW = NC * NS
    n_per_w = N // NW
    CH = 128
    n_ch = n_per_w // CH
    LANES = 16
    mesh = plsc.VectorSubcoreMesh(core_axis_name="c", subcore_axis_name="s")

    @functools.partial(
        pl.kernel,
        out_type=jax.ShapeDtypeStruct((N, D), jnp.float32),
        mesh=mesh,
        scratch_types=[
            pltpu.VMEM((n_per_w,), jnp.int32),
            pltpu.VMEM((n_per_w, D), jnp.float32),
            pltpu.SemaphoreType.DMA,
        ],
        compiler_params=pltpu.CompilerParams(use_tc_tiling_on_sc=False),
    )
    def gather_kernel(table_hbm, idx_hbm, out_hbm, idx_v, rows_v, sem):
        wid = lax.axis_index("s") * NC + lax.axis_index("c")
        base = wid * n_per_w
        pltpu.sync_copy(idx_hbm.at[pl.ds(base, n_per_w)], idx_v)

        def add_offsets(k, carry):
            p = k * LANES + lax.iota(jnp.int32, LANES)
            f = lax.rem(p, F)
            sl = pl.ds(k * LANES, LANES)
            idx_v[sl] = idx_v[sl] + f * V
            return carry

        lax.fori_loop(0, n_per_w // LANES, add_offsets, 0)

        def gather_chunk(j, carry):
            pltpu.async_copy(
                table_hbm.at[idx_v.at[pl.ds(j * CH, CH)]],
                rows_v.at[pl.ds(j * CH, CH)],
                sem,
            ).wait()
            return carry

        lax.fori_loop(0, n_ch, gather_chunk, 0)
        pltpu.sync_copy(rows_v, out_hbm.at[pl.ds(base, n_per_w)])

    return gather_kernel


def _dense_body(x_ref, w1_ref, b1_ref, w2_ref, b2_ref, o_ref):
    h = jnp.dot(x_ref[...], w1_ref[...], preferred_element_type=jnp.float32)
    h = jnp.maximum(h + b1_ref[...], 0.0)
    logits = jnp.dot(h, w2_ref[...], preferred_element_type=jnp.float32)
    logits = logits + b2_ref[...]
    o_ref[...] = 1.0 / (1.0 + jnp.exp(-logits))


def kernel(indices, tables, W1, b1, W2, b2):
    B, F = indices.shape
    _, V, D = tables.shape
    FD, H = W1.shape
    N = B * F

    table_flat = tables.reshape(F * V, D)
    idx_flat = indices.astype(jnp.int32).reshape(N)

    NC, NS = 2, 16
    x = _make_sc_gather(N, D, V, F, NC, NS)(table_flat, idx_flat)
    x = x.reshape(B, FD)

    BB = 512
    out = pl.pallas_call(
        _dense_body,
        grid=(B // BB,),
        in_specs=[
            pl.BlockSpec((BB, FD), lambda i: (i, 0)),
            pl.BlockSpec((FD, H), lambda i: (0, 0)),
            pl.BlockSpec((1, H), lambda i: (0, 0)),
            pl.BlockSpec((H, 1), lambda i: (0, 0)),
            pl.BlockSpec((1, 1), lambda i: (0, 0)),
        ],
        out_specs=pl.BlockSpec((BB, 1), lambda i: (i, 0)),
        out_shape=jax.ShapeDtypeStruct((B, 1), jnp.float32),
    )(x, W1, b1.reshape(1, H), W2, b2.reshape(1, 1))
    return out[:, 0]


# speedup vs baseline: 2.1931x; 1.0078x over previous
import functools

import jax
import jax.numpy as jnp
from jax import lax
from jax.experimental import pallas as pl
from jax.experimental.pallas import tpu as pltpu
from jax.experimental.pallas import tpu_sc as plsc


def _make_sc_gather(N, D, V, F, NC, NS):
    NW = NC * NS
    n_per_w = N // NW
    CH = 128
    n_ch = n_per_w // CH
    LANES = 16
    mesh = plsc.VectorSubcoreMesh(core_axis_name="c", subcore_axis_name="s")

    @functools.partial(
        pl.kernel,
        out_type=jax.ShapeDtypeStruct((N, D), jnp.float32),
        mesh=mesh,
        scratch_types=[
            pltpu.VMEM((n_per_w,), jnp.int32),
            pltpu.VMEM((n_per_w, D), jnp.float32),
            pltpu.SemaphoreType.DMA,
            pltpu.SemaphoreType.DMA,
        ],
        compiler_params=pltpu.CompilerParams(use_tc_tiling_on_sc=False),
    )
    def gather_kernel(table_hbm, idx_hbm, out_hbm, idx_v, rows_v,
                      sem_a, sem_b):
        wid = lax.axis_index("s") * NC + lax.axis_index("c")
        base = wid * n_per_w
        pltpu.sync_copy(idx_hbm.at[pl.ds(base, n_per_w)], idx_v)

        def add_offsets(k, carry):
            p = k * LANES + lax.iota(jnp.int32, LANES)
            f = lax.rem(p, F)
            sl = pl.ds(k * LANES, LANES)
            idx_v[sl] = idx_v[sl] + f * V
            return carry

        lax.fori_loop(0, n_per_w // LANES, add_offsets, 0)

        def fire(j, sem):
            pltpu.async_copy(
                table_hbm.at[idx_v.at[pl.ds(j * CH, CH)]],
                rows_v.at[pl.ds(j * CH, CH)],
                sem,
            )

        def wait(j, sem):
            pltpu.make_async_copy(
                table_hbm.at[idx_v.at[pl.ds(j * CH, CH)]],
                rows_v.at[pl.ds(j * CH, CH)],
                sem,
            ).wait()

        fire(0, sem_a)
        fire(1, sem_b)

        @pl.loop(0, n_ch, step=2)
        def outer(j0):
            for b, sem in enumerate((sem_a, sem_b)):
                j = j0 + b
                wait(j, sem)

                @pl.when(j + 2 < n_ch)
                def _():
                    fire(j + 2, sem)

        pltpu.sync_copy(rows_v, out_hbm.at[pl.ds(base, n_per_w)])

    return gather_kernel


def _dense_body(x_ref, w1_ref, b1_ref, w2_ref, b2_ref, o_ref):
    h = jnp.dot(x_ref[...], w1_ref[...], preferred_element_type=jnp.float32)
    h = jnp.maximum(h + b1_ref[...], 0.0)
    logits = jnp.dot(h, w2_ref[...], preferred_element_type=jnp.float32)
    logits = logits + b2_ref[...]
    o_ref[...] = 1.0 / (1.0 + jnp.exp(-logits))


def kernel(indices, tables, W1, b1, W2, b2):
    B, F = indices.shape
    _, V, D = tables.shape
    FD, H = W1.shape
    N = B * F

    table_flat = tables.reshape(F * V, D)
    idx_flat = indices.astype(jnp.int32).reshape(N)

    NC, NS = 2, 16
    x = _make_sc_gather(N, D, V, F, NC, NS)(table_flat, idx_flat)
    x = x.reshape(B, FD)

    BB = 512
    out = pl.pallas_call(
        _dense_body,
        grid=(B // BB,),
        in_specs=[
            pl.BlockSpec((BB, FD), lambda i: (i, 0)),
            pl.BlockSpec((FD, H), lambda i: (0, 0)),
            pl.BlockSpec((1, H), lambda i: (0, 0)),
            pl.BlockSpec((H, 1), lambda i: (0, 0)),
            pl.BlockSpec((1, 1), lambda i: (0, 0)),
        ],
        out_specs=pl.BlockSpec((BB, 1), lambda i: (i, 0)),
        out_shape=jax.ShapeDtypeStruct((B, 1), jnp.float32),
    )(x, W1, b1.reshape(1, H), W2, b2.reshape(1, 1))
    return out[:, 0]
